# baseline (speedup 1.0000x reference)
import functools

import jax
import jax.numpy as jnp
import numpy as np
from jax import lax
from jax.experimental import pallas as pl
from jax.experimental.pallas import tpu as pltpu
from jax.experimental.pallas import tpu_sc as plsc

_L = 16
_H = 224
_W = 224
_HW = _H * _W
_NIMG = 192
_NWORKERS = 32
_IMGS_PER_WORKER = _NIMG // _NWORKERS
_ROW_VECS = _W // 16
_QUNROLL = 8
_TC_BLK = 32


def _sc_hist_body(x_hbm, out_hbm, buf, qbuf, hist, sem):
    wid = lax.axis_index("s") * 2 + lax.axis_index("c")
    img0 = wid * _IMGS_PER_WORKER

    iota = lax.iota(jnp.int32, 16)
    ones = jnp.ones((16,), jnp.float32)
    zeros = jnp.zeros((16,), jnp.float32)
    lane0 = iota == 0
    lane15 = iota == 15
    rotr_idx = (iota + 15) & 15
    rotl_idx = (iota + 1) & 15

    def _rot(x, idx):
        return lax.gather(
            x, idx[:, None],
            dimension_numbers=lax.GatherDimensionNumbers(
                offset_dims=(), collapsed_slice_dims=(0,),
                start_index_map=(0,)),
            slice_sizes=(1,),
            mode=lax.GatherScatterMode.PROMISE_IN_BOUNDS,
        )

    pending = pltpu.async_copy(x_hbm.at[img0], buf, sem)

    for k in range(_IMGS_PER_WORKER):
        pending.wait()

        def _quant_blk(i, _):
            base = i * (16 * _QUNROLL)
            vals = [buf[pl.ds(base + u * 16, 16)] for u in range(_QUNROLL)]
            qs = [(v * (_L - 1)).astype(jnp.int32) for v in vals]
            for u in range(_QUNROLL):
                qbuf[pl.ds(base + u * 16, 16)] = qs[u]
            return _
        lax.fori_loop(0, _HW // (16 * _QUNROLL), _quant_blk, None)

        if k + 1 < _IMGS_PER_WORKER:
            pending = pltpu.async_copy(x_hbm.at[img0 + (k + 1)], buf, sem)

        def _zero(i, _):
            hist[pl.ds(i * 16, 16)] = zeros
            return _
        lax.fori_loop(0, 256 // 16, _zero, None)

        def _rowpair(i, _):
            rowa = (2 * i) * _W
            rowbb = rowa + _W
            preva = jnp.where(i == 0, (_H - 1) * _W, rowa - _W)

            def _load(jb):
                cb = jb * 16
                return (
                    qbuf[pl.ds(rowa + cb, 16)],
                    qbuf[pl.ds(rowbb + cb, 16)],
                    qbuf[pl.ds(preva + cb, 16)],
                )

            def _rots(c):
                a, bb, p = c
                return (_rot(a, rotr_idx), _rot(bb, rotr_idx),
                        _rot(p, rotr_idx), _rot(a, rotl_idx),
                        _rot(p, rotl_idx))

            cs = {13: _load(13), 0: _load(0), 1: _load(1)}
            rs = {13: _rots(cs[13]), 0: _rots(cs[0]), 1: _rots(cs[1])}

            for jb in range(_ROW_VECS):
                a, bb, p = cs[jb]
                rr_a, rr_b, rr_p, rl_a, rl_p = rs[jb]
                rprev = rs[13 if jb == 0 else jb - 1]
                rnext = rs[0 if jb == _ROW_VECS - 1 else jb + 1]
                n0a = jnp.where(lane0, rprev[0], rr_a)
                n0b = jnp.where(lane0, rprev[1], rr_b)
                n135a = jnp.where(lane0, rprev[2], rr_p)
                n45a = jnp.where(lane15, rnext[4], rl_p)
                n45b = jnp.where(lane15, rnext[3], rl_a)
                q16a = a * _L
                q16b = bb * _L
                for idx in (
                    q16a + n0a, q16a + n45a, q16a + p, q16a + n135a,
                    q16b + n0b, q16b + n45b, q16b + a, q16b + n0a,
                ):
                    plsc.addupdate_scatter(hist, [idx], ones)
                nxt = jb + 2
                if nxt < _ROW_VECS - 1:
                    cs[nxt] = _load(nxt)
                    rs[nxt] = _rots(cs[nxt])
            return _

        lax.fori_loop(0, _H // 2, _rowpair, None)

        pltpu.sync_copy(hist, out_hbm.at[img0 + k])


_sc_hist = functools.partial(
    pl.kernel,
    out_type=jax.ShapeDtypeStruct((_NIMG, _L * _L), jnp.float32),
    mesh=plsc.VectorSubcoreMesh(core_axis_name="c", subcore_axis_name="s"),
    scratch_types=[
        pltpu.VMEM((_HW,), jnp.float32),
        pltpu.VMEM((_HW,), jnp.int32),
        pltpu.VMEM((_L * _L,), jnp.float32),
        pltpu.SemaphoreType.DMA,
    ],
    compiler_params=pltpu.CompilerParams(needs_layout_passes=False),
)(_sc_hist_body)


def _tc_entropy_body(hist_ref, out_ref):
    h = hist_ref[...]
    g = h / jnp.sum(h, axis=-1, keepdims=True)
    e = -jnp.sum(g * jnp.log(g + 1e-10), axis=-1)
    out_ref[...] = jnp.broadcast_to(e[:, :, None], (_TC_BLK, _H, _W))


def kernel(x):
    b, c, h, w = x.shape
    xr = x.reshape(_NIMG, _HW)
    hist = _sc_hist(xr).reshape(_NIMG, 1, _L * _L)
    ent = pl.pallas_call(
        _tc_entropy_body,
        grid=(_NIMG // _TC_BLK,),
        in_specs=[pl.BlockSpec((_TC_BLK, 1, _L * _L), lambda i: (i, 0, 0))],
        out_specs=pl.BlockSpec((_TC_BLK, _H, _W), lambda i: (i, 0, 0)),
        out_shape=jax.ShapeDtypeStruct((_NIMG, _H, _W), jnp.float32),
    )(hist)
    return ent.reshape(b, c, h, w)


# speedup vs baseline: 1.0764x; 1.0764x over previous
import functools

import jax
import jax.numpy as jnp
from jax import lax
from jax.experimental import pallas as pl
from jax.experimental.pallas import tpu as pltpu
from jax.experimental.pallas import tpu_sc as plsc

_L = 16
_H = 224
_W = 224
_HW = _H * _W
_NIMG = 192
_NWORKERS = 32
_IMGS_PER_WORKER = _NIMG // _NWORKERS
_ROW_VECS = _W // 16
_QUNROLL = 8
_LN_TOTAL = 12.20958646482997


def _sc_hist_body(x_hbm, out_hbm, buf, qbuf, hist, ebuf, sem):
    wid = lax.axis_index("s") * 2 + lax.axis_index("c")
    img0 = wid * _IMGS_PER_WORKER

    iota = lax.iota(jnp.int32, 16)
    ones = jnp.ones((16,), jnp.float32)
    zeros = jnp.zeros((16,), jnp.float32)
    ione = jnp.ones((16,), jnp.int32)
    izero = jnp.zeros((16,), jnp.int32)
    cm1_first = jnp.where(iota == 0, jnp.full((16,), 223, jnp.int32), iota - 1)
    cp1_last = jnp.where(
        iota == 15, jnp.full((16,), -15, jnp.int32), iota + 1
    ) + (_W - 16)

    pending = pltpu.async_copy(x_hbm.at[img0], buf, sem)

    for k in range(_IMGS_PER_WORKER):
        pending.wait()

        def _quant_blk(i, _):
            base = i * (16 * _QUNROLL)
            vals = [buf[pl.ds(base + u * 16, 16)] for u in range(_QUNROLL)]
            qs = [(v * (_L - 1)).astype(jnp.int32) for v in vals]
            for u in range(_QUNROLL):
                qbuf[pl.ds(base + u * 16, 16)] = qs[u]
            return _
        lax.fori_loop(0, _HW // (16 * _QUNROLL), _quant_blk, None)

        if k + 1 < _IMGS_PER_WORKER:
            pending = pltpu.async_copy(x_hbm.at[img0 + (k + 1)], buf, sem)

        def _zero(i, _):
            hist[pl.ds(i * 16, 16)] = zeros
            return _
        lax.fori_loop(0, 256 // 16, _zero, None)

        def _rowpair(i, _):
            rowa = (2 * i) * _W
            rowbb = rowa + _W
            preva = jnp.where(i == 0, (_H - 1) * _W, rowa - _W)

            def _load(jb):
                cb = jb * 16
                qa = qbuf[pl.ds(rowa + cb, 16)]
                qb = qbuf[pl.ds(rowbb + cb, 16)]
                n90a = qbuf[pl.ds(preva + cb, 16)]
                if jb == 0:
                    n0a = plsc.load_gather(qbuf, [rowa + cm1_first])
                    n0b = plsc.load_gather(qbuf, [rowbb + cm1_first])
                    n135a = plsc.load_gather(qbuf, [preva + cm1_first])
                    n135b = plsc.load_gather(qbuf, [rowa + cm1_first])
                else:
                    n0a = qbuf[pl.ds(rowa + cb - 1, 16)]
                    n0b = qbuf[pl.ds(rowbb + cb - 1, 16)]
                    n135a = qbuf[pl.ds(preva + cb - 1, 16)]
                    n135b = qbuf[pl.ds(rowa + cb - 1, 16)]
                if jb == _ROW_VECS - 1:
                    n45a = plsc.load_gather(qbuf, [preva + cp1_last])
                    n45b = plsc.load_gather(qbuf, [rowa + cp1_last])
                else:
                    n45a = qbuf[pl.ds(preva + cb + 1, 16)]
                    n45b = qbuf[pl.ds(rowa + cb + 1, 16)]
                return qa, qb, n90a, n0a, n0b, n45a, n45b, n135a, n135b

            def _compute(vals):
                qa, qb, n90a, n0a, n0b, n45a, n45b, n135a, n135b = vals
                q16a = qa * _L
                q16b = qb * _L
                return (
                    q16a + n0a, q16a + n45a, q16a + n90a, q16a + n135a,
                    q16b + n0b, q16b + n45b, q16b + qa, q16b + n135b,
                )

            def _scatter(idxs):
                for idx in idxs:
                    plsc.addupdate_scatter(hist, [idx], ones)

            vals = _load(0)
            idxs = _compute(vals)
            vals = _load(1)
            for jb in range(2, _ROW_VECS):
                _scatter(idxs)
                idxs = _compute(vals)
                vals = _load(jb)
            _scatter(idxs)
            idxs = _compute(vals)
            _scatter(idxs)
            return _

        lax.fori_loop(0, _H // 2, _rowpair, None)

        acc = zeros
        for v in range(256 // 16):
            h = hist[pl.ds(v * 16, 16)]
            hi = h.astype(jnp.int32)
            e = izero
            for p in range(1, 18):
                e = e + jnp.where(hi >= (1 << p), ione, izero)
            m = h / (ione << e).astype(jnp.float32)
            t = m - 1.0
            z = t / (t + 2.0)
            z2 = z * z
            ln_m = 2.0 * z * (1.0 + z2 * (
                0.3333333333 + z2 * (0.2 + z2 * 0.1428571429)))
            ln_h = e.astype(jnp.float32) * 0.6931471805599453 + ln_m
            acc = acc + h * ln_h
        total = float(4 * _HW)
        s = jnp.sum(acc)
        ebuf[...] = jnp.full((16,), _LN_TOTAL, jnp.float32) - s * (1.0 / total)
        pltpu.sync_copy(ebuf, out_hbm.at[img0 + k])


_sc_hist = functools.partial(
    pl.kernel,
    out_type=jax.ShapeDtypeStruct((_NIMG, 16), jnp.float32),
    mesh=plsc.VectorSubcoreMesh(core_axis_name="c", subcore_axis_name="s"),
    scratch_types=[
        pltpu.VMEM((_HW,), jnp.float32),
        pltpu.VMEM((_HW,), jnp.int32),
        pltpu.VMEM((_L * _L,), jnp.float32),
        pltpu.VMEM((16,), jnp.float32),
        pltpu.SemaphoreType.DMA,
    ],
    compiler_params=pltpu.CompilerParams(needs_layout_passes=False),
)(_sc_hist_body)


def kernel(x):
    b, c, h, w = x.shape
    xr = x.reshape(_NIMG, _HW)
    ent = _sc_hist(xr)[:, 0]
    return jnp.broadcast_to(ent.reshape(b, c, 1, 1), (b, c, h, w))


# speedup vs baseline: 1.1068x; 1.0282x over previous
import functools

import jax
import jax.numpy as jnp
from jax import lax
from jax.experimental import pallas as pl
from jax.experimental.pallas import tpu as pltpu
from jax.experimental.pallas import tpu_sc as plsc

_L = 16
_H = 224
_W = 224
_HW = _H * _W
_NIMG = 192
_NWORKERS = 32
_IMGS_PER_WORKER = _NIMG // _NWORKERS
_ROW_VECS = _W // 16
_QUNROLL = 8
_LN_TOTAL = 12.20958646482997


def _sc_hist_body(x_hbm, out_hbm, buf, qbuf, hist, ebuf, sem):
    wid = lax.axis_index("s") * 2 + lax.axis_index("c")
    img0 = wid * _IMGS_PER_WORKER

    iota = lax.iota(jnp.int32, 16)
    ones = jnp.ones((16,), jnp.float32)
    zeros = jnp.zeros((16,), jnp.float32)
    ione = jnp.ones((16,), jnp.int32)
    izero = jnp.zeros((16,), jnp.int32)
    cm1_first = jnp.where(iota == 0, jnp.full((16,), 223, jnp.int32), iota - 1)
    cp1_last = jnp.where(
        iota == 15, jnp.full((16,), -15, jnp.int32), iota + 1
    ) + (_W - 16)

    pltpu.async_copy(x_hbm.at[img0], buf, sem)

    def _image(k, _):
        img = img0 + k
        pltpu.make_async_copy(x_hbm.at[img], buf, sem).wait()

        def _quant_blk(i, _):
            base = i * (16 * _QUNROLL)
            vals = [buf[pl.ds(base + u * 16, 16)] for u in range(_QUNROLL)]
            qs = [(v * (_L - 1)).astype(jnp.int32) for v in vals]
            for u in range(_QUNROLL):
                qbuf[pl.ds(base + u * 16, 16)] = qs[u]
            return _
        lax.fori_loop(0, _HW // (16 * _QUNROLL), _quant_blk, None)

        nxt = jnp.minimum(img + 1, img0 + _IMGS_PER_WORKER - 1)
        pltpu.async_copy(x_hbm.at[nxt], buf, sem)

        def _zero(i, _):
            hist[pl.ds(i * 16, 16)] = zeros
            return _
        lax.fori_loop(0, 256 // 16, _zero, None)

        def _rowpair(i, _):
            rowa = (2 * i) * _W
            rowbb = rowa + _W
            preva = jnp.where(i == 0, (_H - 1) * _W, rowa - _W)

            def _load(jb):
                cb = jb * 16
                qa = qbuf[pl.ds(rowa + cb, 16)]
                qb = qbuf[pl.ds(rowbb + cb, 16)]
                n90a = qbuf[pl.ds(preva + cb, 16)]
                if jb == 0:
                    n0a = plsc.load_gather(qbuf, [rowa + cm1_first])
                    n0b = plsc.load_gather(qbuf, [rowbb + cm1_first])
                    n135a = plsc.load_gather(qbuf, [preva + cm1_first])
                    n135b = plsc.load_gather(qbuf, [rowa + cm1_first])
                else:
                    n0a = qbuf[pl.ds(rowa + cb - 1, 16)]
                    n0b = qbuf[pl.ds(rowbb + cb - 1, 16)]
                    n135a = qbuf[pl.ds(preva + cb - 1, 16)]
                    n135b = qbuf[pl.ds(rowa + cb - 1, 16)]
                if jb == _ROW_VECS - 1:
                    n45a = plsc.load_gather(qbuf, [preva + cp1_last])
                    n45b = plsc.load_gather(qbuf, [rowa + cp1_last])
                else:
                    n45a = qbuf[pl.ds(preva + cb + 1, 16)]
                    n45b = qbuf[pl.ds(rowa + cb + 1, 16)]
                return qa, qb, n90a, n0a, n0b, n45a, n45b, n135a, n135b

            def _compute(vals):
                qa, qb, n90a, n0a, n0b, n45a, n45b, n135a, n135b = vals
                q16a = qa * _L
                q16b = qb * _L
                return (
                    q16a + n0a, q16a + n45a, q16a + n90a, q16a + n135a,
                    q16b + n0b, q16b + n45b, q16b + qa, q16b + n135b,
                )

            def _scatter(idxs):
                for idx in idxs:
                    plsc.addupdate_scatter(hist, [idx], ones)

            vals = _load(0)
            idxs = _compute(vals)
            vals = _load(1)
            for jb in range(2, _ROW_VECS):
                _scatter(idxs)
                idxs = _compute(vals)
                vals = _load(jb)
            _scatter(idxs)
            idxs = _compute(vals)
            _scatter(idxs)
            return _

        lax.fori_loop(0, _H // 2, _rowpair, None)

        acc = zeros
        for v in range(256 // 16):
            h = hist[pl.ds(v * 16, 16)]
            hi = h.astype(jnp.int32)
            e = izero
            for p in range(1, 18):
                e = e + jnp.where(hi >= (1 << p), ione, izero)
            m = h / (ione << e).astype(jnp.float32)
            t = m - 1.0
            z = t / (t + 2.0)
            z2 = z * z
            ln_m = 2.0 * z * (1.0 + z2 * (
                0.3333333333 + z2 * (0.2 + z2 * 0.1428571429)))
            ln_h = e.astype(jnp.float32) * 0.6931471805599453 + ln_m
            acc = acc + h * ln_h
        total = float(4 * _HW)
        s = jnp.sum(acc)
        ebuf[...] = jnp.full((16,), _LN_TOTAL, jnp.float32) - s * (1.0 / total)
        pltpu.sync_copy(ebuf, out_hbm.at[img])
        return _

    lax.fori_loop(0, _IMGS_PER_WORKER, _image, None)
    pltpu.make_async_copy(x_hbm.at[img0], buf, sem).wait()


_sc_hist = functools.partial(
    pl.kernel,
    out_type=jax.ShapeDtypeStruct((_NIMG, 16), jnp.float32),
    mesh=plsc.VectorSubcoreMesh(core_axis_name="c", subcore_axis_name="s"),
    scratch_types=[
        pltpu.VMEM((_HW,), jnp.float32),
        pltpu.VMEM((_HW,), jnp.int32),
        pltpu.VMEM((_L * _L,), jnp.float32),
        pltpu.VMEM((16,), jnp.float32),
        pltpu.SemaphoreType.DMA,
    ],
    compiler_params=pltpu.CompilerParams(needs_layout_passes=False),
)(_sc_hist_body)


def kernel(x):
    b, c, h, w = x.shape
    xr = x.reshape(_NIMG, _HW)
    ent = _sc_hist(xr)[:, 0]
    return jnp.broadcast_to(ent.reshape(b, c, 1, 1), (b, c, h, w))


# speedup vs baseline: 1.1288x; 1.0199x over previous
import functools

import jax
import jax.numpy as jnp
from jax import lax
from jax.experimental import pallas as pl
from jax.experimental.pallas import tpu as pltpu
from jax.experimental.pallas import tpu_sc as plsc

_L = 16
_H = 224
_W = 224
_HW = _H * _W
_NIMG = 192
_NWORKERS = 32
_IMGS_PER_WORKER = _NIMG // _NWORKERS
_ROW_VECS = _W // 16
_QUNROLL = 8
_LN_TOTAL = 12.20958646482997


def _sc_hist_body(x_hbm, out_hbm, buf, qbuf, hist, ebuf, sem):
    wid = lax.axis_index("s") * 2 + lax.axis_index("c")
    img0 = wid * _IMGS_PER_WORKER

    iota = lax.iota(jnp.int32, 16)
    ones = jnp.ones((16,), jnp.float32)
    zeros = jnp.zeros((16,), jnp.float32)
    ione = jnp.ones((16,), jnp.int32)
    izero = jnp.zeros((16,), jnp.int32)
    cm1_first = jnp.where(iota == 0, jnp.full((16,), 223, jnp.int32), iota - 1)
    cp1_last = jnp.where(
        iota == 15, jnp.full((16,), -15, jnp.int32), iota + 1
    ) + (_W - 16)

    pltpu.async_copy(x_hbm.at[img0], buf, sem)

    def _image(k, _):
        img = img0 + k
        pltpu.make_async_copy(x_hbm.at[img], buf, sem).wait()

        def _quant_blk(i, _):
            base = i * (16 * _QUNROLL)
            vals = [buf[pl.ds(base + u * 16, 16)] for u in range(_QUNROLL)]
            qs = [(v * (_L - 1)).astype(jnp.int32) for v in vals]
            for u in range(_QUNROLL):
                qbuf[pl.ds(base + u * 16, 16)] = qs[u]
            return _
        lax.fori_loop(0, _HW // (16 * _QUNROLL), _quant_blk, None)

        nxt = jnp.minimum(img + 1, img0 + _IMGS_PER_WORKER - 1)
        pltpu.async_copy(x_hbm.at[nxt], buf, sem)

        def _zero(i, _):
            hist[pl.ds(i * 16, 16)] = zeros
            return _
        lax.fori_loop(0, 256 // 16, _zero, None)

        def _rowquad(i, _):
            rows = [(4 * i + r) * _W for r in range(4)]
            prev0 = jnp.where(i == 0, (_H - 1) * _W, rows[0] - _W)
            bases = [prev0] + rows

            def _load(jb):
                cb = jb * 16
                qs = [qbuf[pl.ds(rb + cb, 16)] for rb in rows]
                n90_0 = qbuf[pl.ds(prev0 + cb, 16)]
                if jb == 0:
                    lefts = [
                        plsc.load_gather(qbuf, [rb + cm1_first])
                        for rb in bases
                    ]
                else:
                    lefts = [
                        qbuf[pl.ds(rb + cb - 1, 16)] for rb in bases
                    ]
                if jb == _ROW_VECS - 1:
                    rights = [
                        plsc.load_gather(qbuf, [rb + cp1_last])
                        for rb in bases[:4]
                    ]
                else:
                    rights = [
                        qbuf[pl.ds(rb + cb + 1, 16)] for rb in bases[:4]
                    ]
                return qs, n90_0, lefts, rights

            def _compute(vals):
                qs, n90_0, lefts, rights = vals
                ups = [n90_0] + qs[:3]
                idxs = []
                for r in range(4):
                    q16 = qs[r] * _L
                    idxs += [
                        q16 + lefts[r + 1],
                        q16 + rights[r],
                        q16 + ups[r],
                        q16 + lefts[r],
                    ]
                return idxs

            def _scatter(idxs):
                for idx in idxs:
                    plsc.addupdate_scatter(hist, [idx], ones)

            vals = _load(0)
            idxs = _compute(vals)
            vals = _load(1)
            for jb in range(2, _ROW_VECS):
                _scatter(idxs)
                idxs = _compute(vals)
                vals = _load(jb)
            _scatter(idxs)
            idxs = _compute(vals)
            _scatter(idxs)
            return _

        lax.fori_loop(0, _H // 4, _rowquad, None)

        acc = zeros
        for v in range(256 // 16):
            h = hist[pl.ds(v * 16, 16)]
            hi = h.astype(jnp.int32)
            e = izero
            for p in range(1, 18):
                e = e + jnp.where(hi >= (1 << p), ione, izero)
            m = h / (ione << e).astype(jnp.float32)
            t = m - 1.0
            z = t / (t + 2.0)
            z2 = z * z
            ln_m = 2.0 * z * (1.0 + z2 * (
                0.3333333333 + z2 * (0.2 + z2 * 0.1428571429)))
            ln_h = e.astype(jnp.float32) * 0.6931471805599453 + ln_m
            acc = acc + h * ln_h
        total = float(4 * _HW)
        s = jnp.sum(acc)
        ebuf[...] = jnp.full((16,), _LN_TOTAL, jnp.float32) - s * (1.0 / total)
        pltpu.sync_copy(ebuf, out_hbm.at[img])
        return _

    lax.fori_loop(0, _IMGS_PER_WORKER, _image, None)
    pltpu.make_async_copy(x_hbm.at[img0], buf, sem).wait()


_sc_hist = functools.partial(
    pl.kernel,
    out_type=jax.ShapeDtypeStruct((_NIMG, 16), jnp.float32),
    mesh=plsc.VectorSubcoreMesh(core_axis_name="c", subcore_axis_name="s"),
    scratch_types=[
        pltpu.VMEM((_HW,), jnp.float32),
        pltpu.VMEM((_HW,), jnp.int32),
        pltpu.VMEM((_L * _L,), jnp.float32),
        pltpu.VMEM((16,), jnp.float32),
        pltpu.SemaphoreType.DMA,
    ],
    compiler_params=pltpu.CompilerParams(needs_layout_passes=False),
)(_sc_hist_body)


def kernel(x):
    b, c, h, w = x.shape
    xr = x.reshape(_NIMG, _HW)
    ent = _sc_hist(xr)[:, 0]
    return jnp.broadcast_to(ent.reshape(b, c, 1, 1), (b, c, h, w))
